# Initial kernel scaffold; baseline (speedup 1.0000x reference)
import functools

import jax
import jax.numpy as jnp
from jax import lax
from jax.experimental import pallas as pl
from jax.experimental.pallas import tpu as pltpu
from jax.experimental.pallas import tpu_sc as plsc

D = 64
LANES = 16


@functools.lru_cache(maxsize=None)
def _make_gather_sum(N, L, C):
    NC, NS = 2, 16
    NW = NC * NS
    assert N % (NW * C) == 0
    rows_per_w = N // NW
    steps = rows_per_w // C
    mesh = plsc.VectorSubcoreMesh(core_axis_name="c", subcore_axis_name="s")

    @functools.partial(
        pl.kernel,
        out_type=jax.ShapeDtypeStruct((N, D), jnp.float32),
        mesh=mesh,
        scratch_types=[
            pltpu.VMEM((C * L,), jnp.int32),
            pltpu.VMEM((C * L, D), jnp.float32),
            pltpu.VMEM((C, D), jnp.float32),
            pltpu.SemaphoreType.DMA,
        ],
    )
    def gather_sum(idx_hbm, emb_hbm, out_hbm, idx_v, rows_v, out_v, sem):
        wid = lax.axis_index("s") * NC + lax.axis_index("c")
        row0_w = wid * rows_per_w

        def step(i, carry):
            row0 = row0_w + i * C
            pltpu.sync_copy(idx_hbm.at[pl.ds(row0 * L, C * L)], idx_v)
            pltpu.async_copy(emb_hbm.at[idx_v], rows_v, sem).wait()

            def reduce_row(r, carry2):
                accs = [rows_v[r * L, pl.ds(k * LANES, LANES)]
                        for k in range(D // LANES)]
                for j in range(1, L):
                    accs = [a + rows_v[r * L + j, pl.ds(k * LANES, LANES)]
                            for k, a in enumerate(accs)]
                for k, a in enumerate(accs):
                    out_v[r, pl.ds(k * LANES, LANES)] = a
                return carry2

            lax.fori_loop(0, C, reduce_row, 0)
            pltpu.sync_copy(out_v, out_hbm.at[pl.ds(row0, C)])
            return carry

        lax.fori_loop(0, steps, step, 0)

    return gather_sum


def _finish_body(tok_ref, sums_ref, w_ref, b_ref, out_ref):
    tok = tok_ref[...]
    cnt = jnp.sum((tok != 0).astype(jnp.float32), axis=1, keepdims=True)
    inv = 1.0 / jnp.maximum(cnt, 1.0)
    mean = sums_ref[...] * inv
    acc = lax.dot_general(mean, w_ref[...], (((1,), (1,)), ((), ())),
                          preferred_element_type=jnp.float32)
    out_ref[...] = jnp.maximum(acc + b_ref[...], 0.0)


def _finish(tok, sums, W, b, block_rows=2048):
    N, L = tok.shape
    assert N % block_rows == 0
    return pl.pallas_call(
        _finish_body,
        grid=(N // block_rows,),
        in_specs=[
            pl.BlockSpec((block_rows, L), lambda i: (i, 0)),
            pl.BlockSpec((block_rows, D), lambda i: (i, 0)),
            pl.BlockSpec((D, D), lambda i: (0, 0)),
            pl.BlockSpec((1, D), lambda i: (0, 0)),
        ],
        out_specs=pl.BlockSpec((block_rows, D), lambda i: (i, 0)),
        out_shape=jax.ShapeDtypeStruct((N, D), jnp.float32),
    )(tok, sums, W, b.reshape(1, D))


def kernel(sample, emb, W, b):
    L = sample.shape[-1]
    flat = sample.reshape(-1, L).astype(jnp.int32)
    N = flat.shape[0]
    sums = _make_gather_sum(N, L, 16)(flat.reshape(-1), emb)
    out = _finish(flat, sums, W, b)
    return out.reshape(sample.shape[:-1] + (D,))


# speedup vs baseline: 3.1211x; 3.1211x over previous
import functools

import jax
import jax.numpy as jnp
from jax import lax
from jax.experimental import pallas as pl
from jax.experimental.pallas import tpu as pltpu
from jax.experimental.pallas import tpu_sc as plsc

D = 64
LANES = 16


@functools.lru_cache(maxsize=None)
def _make_gather_sum(N, L, C):
    NC, NS = 2, 16
    NW = NC * NS
    assert N % (NW * C) == 0
    rows_per_w = N // NW
    steps = rows_per_w // C
    mesh = plsc.VectorSubcoreMesh(core_axis_name="c", subcore_axis_name="s")

    @functools.partial(
        pl.kernel,
        out_type=jax.ShapeDtypeStruct((N, D), jnp.float32),
        mesh=mesh,
        scratch_types=[
            pltpu.VMEM((C * L,), jnp.int32),
            pltpu.VMEM((C * L, D), jnp.float32),
            pltpu.VMEM((C, D), jnp.float32),
            pltpu.SemaphoreType.DMA,
        ],
        compiler_params=pltpu.CompilerParams(use_tc_tiling_on_sc=False),
    )
    def gather_sum(idx_hbm, emb_hbm, out_hbm, idx_v, rows_v, out_v, sem):
        wid = lax.axis_index("s") * NC + lax.axis_index("c")
        row0_w = wid * rows_per_w

        def step(i, carry):
            row0 = row0_w + i * C
            pltpu.sync_copy(idx_hbm.at[pl.ds(row0 * L, C * L)], idx_v)
            pltpu.async_copy(emb_hbm.at[idx_v], rows_v, sem).wait()

            def reduce_row(r, carry2):
                accs = [rows_v[r * L, pl.ds(k * LANES, LANES)]
                        for k in range(D // LANES)]
                for j in range(1, L):
                    accs = [a + rows_v[r * L + j, pl.ds(k * LANES, LANES)]
                            for k, a in enumerate(accs)]
                for k, a in enumerate(accs):
                    out_v[r, pl.ds(k * LANES, LANES)] = a
                return carry2

            lax.fori_loop(0, C, reduce_row, 0)
            pltpu.sync_copy(out_v, out_hbm.at[pl.ds(row0, C)])
            return carry

        lax.fori_loop(0, steps, step, 0)

    return gather_sum


def _finish_body(tok_ref, sums_ref, w_ref, b_ref, out_ref):
    tok = tok_ref[...]
    cnt = jnp.sum((tok != 0).astype(jnp.float32), axis=1, keepdims=True)
    inv = 1.0 / jnp.maximum(cnt, 1.0)
    mean = sums_ref[...] * inv
    acc = lax.dot_general(mean, w_ref[...], (((1,), (1,)), ((), ())),
                          preferred_element_type=jnp.float32)
    out_ref[...] = jnp.maximum(acc + b_ref[...], 0.0)


def _finish(tok, sums, W, b, block_rows=2048):
    N, L = tok.shape
    assert N % block_rows == 0
    return pl.pallas_call(
        _finish_body,
        grid=(N // block_rows,),
        in_specs=[
            pl.BlockSpec((block_rows, L), lambda i: (i, 0)),
            pl.BlockSpec((block_rows, D), lambda i: (i, 0)),
            pl.BlockSpec((D, D), lambda i: (0, 0)),
            pl.BlockSpec((1, D), lambda i: (0, 0)),
        ],
        out_specs=pl.BlockSpec((block_rows, D), lambda i: (i, 0)),
        out_shape=jax.ShapeDtypeStruct((N, D), jnp.float32),
    )(tok, sums, W, b.reshape(1, D))


def kernel(sample, emb, W, b):
    L = sample.shape[-1]
    flat = sample.reshape(-1, L).astype(jnp.int32)
    N = flat.shape[0]
    sums = _make_gather_sum(N, L, 16)(flat.reshape(-1), emb)
    out = _finish(flat, sums, W, b)
    return out.reshape(sample.shape[:-1] + (D,))


# speedup vs baseline: 3.5153x; 1.1263x over previous
import functools

import jax
import jax.numpy as jnp
from jax import lax
from jax.experimental import pallas as pl
from jax.experimental.pallas import tpu as pltpu
from jax.experimental.pallas import tpu_sc as plsc

D = 64
LANES = 16


@functools.lru_cache(maxsize=None)
def _make_gather_sum(N, L, C):
    NC, NS = 2, 16
    NW = NC * NS
    assert N % (NW * C) == 0
    rows_per_w = N // NW
    steps = rows_per_w // C
    assert steps % 2 == 0
    toks_w = rows_per_w * L
    CL = C * L
    mesh = plsc.VectorSubcoreMesh(core_axis_name="c", subcore_axis_name="s")

    @functools.partial(
        pl.kernel,
        out_type=jax.ShapeDtypeStruct((N, D), jnp.float32),
        mesh=mesh,
        scratch_types=[
            pltpu.VMEM((toks_w,), jnp.int32),
            pltpu.VMEM((CL, D), jnp.float32),
            pltpu.VMEM((CL, D), jnp.float32),
            pltpu.VMEM((C, D), jnp.float32),
            pltpu.VMEM((C, D), jnp.float32),
            pltpu.SemaphoreType.DMA,
            pltpu.SemaphoreType.DMA,
        ],
        compiler_params=pltpu.CompilerParams(use_tc_tiling_on_sc=False),
    )
    def gather_sum(idx_hbm, emb_hbm, out_hbm, idx_v, rows0, rows1,
                   out0, out1, sem0, sem1):
        wid = lax.axis_index("s") * NC + lax.axis_index("c")
        row0_w = wid * rows_per_w
        pltpu.sync_copy(idx_hbm.at[pl.ds(row0_w * L, toks_w)], idx_v)

        def gather(chunk, rows, sem):
            pltpu.async_copy(
                emb_hbm.at[idx_v.at[pl.ds(chunk * CL, CL)]], rows, sem)

        def wait_gather(rows, sem):
            pltpu.make_async_copy(
                emb_hbm.at[idx_v.at[pl.ds(0, CL)]], rows, sem).wait()

        def reduce_chunk(chunk, rows, outv):
            def reduce_row(r, carry2):
                accs = [rows[r * L, pl.ds(k * LANES, LANES)]
                        for k in range(D // LANES)]
                for j in range(1, L):
                    accs = [a + rows[r * L + j, pl.ds(k * LANES, LANES)]
                            for k, a in enumerate(accs)]
                for k, a in enumerate(accs):
                    outv[r, pl.ds(k * LANES, LANES)] = a
                return carry2

            lax.fori_loop(0, C, reduce_row, 0)
            pltpu.sync_copy(outv, out_hbm.at[pl.ds(row0_w + chunk * C, C)])

        gather(0, rows0, sem0)

        def body(i, carry):
            c0 = 2 * i
            gather(c0 + 1, rows1, sem1)
            wait_gather(rows0, sem0)
            reduce_chunk(c0, rows0, out0)
            gather(jnp.minimum(c0 + 2, steps - 1), rows0, sem0)
            wait_gather(rows1, sem1)
            reduce_chunk(c0 + 1, rows1, out1)
            return carry

        lax.fori_loop(0, steps // 2, body, 0)
        wait_gather(rows0, sem0)

    return gather_sum


def _finish_body(tok_ref, sums_ref, w_ref, b_ref, out_ref):
    tok = tok_ref[...]
    cnt = jnp.sum((tok != 0).astype(jnp.float32), axis=1, keepdims=True)
    inv = 1.0 / jnp.maximum(cnt, 1.0)
    mean = sums_ref[...] * inv
    acc = lax.dot_general(mean, w_ref[...], (((1,), (1,)), ((), ())),
                          preferred_element_type=jnp.float32)
    out_ref[...] = jnp.maximum(acc + b_ref[...], 0.0)


def _finish(tok, sums, W, b, block_rows=2048):
    N, L = tok.shape
    assert N % block_rows == 0
    return pl.pallas_call(
        _finish_body,
        grid=(N // block_rows,),
        in_specs=[
            pl.BlockSpec((block_rows, L), lambda i: (i, 0)),
            pl.BlockSpec((block_rows, D), lambda i: (i, 0)),
            pl.BlockSpec((D, D), lambda i: (0, 0)),
            pl.BlockSpec((1, D), lambda i: (0, 0)),
        ],
        out_specs=pl.BlockSpec((block_rows, D), lambda i: (i, 0)),
        out_shape=jax.ShapeDtypeStruct((N, D), jnp.float32),
    )(tok, sums, W, b.reshape(1, D))


def kernel(sample, emb, W, b):
    L = sample.shape[-1]
    flat = sample.reshape(-1, L).astype(jnp.int32)
    N = flat.shape[0]
    sums = _make_gather_sum(N, L, 8)(flat.reshape(-1), emb)
    out = _finish(flat, sums, W, b)
    return out.reshape(sample.shape[:-1] + (D,))
